# baseline (speedup 1.0000x reference)
import functools

import jax
import jax.numpy as jnp
from jax import lax
from jax.experimental import pallas as pl
from jax.experimental.pallas import tpu as pltpu
from jax.experimental.pallas import tpu_sc as plsc

NUM_CHARS = 1000
EMB_DIM = 64


def _mm_body(emb_ref, w_ref, b_ref, m_ref):
    m_ref[...] = lax.dot_general(
        emb_ref[...], w_ref[...],
        dimension_numbers=(((1,), (1,)), ((), ())),
        preferred_element_type=jnp.float32,
    ) + b_ref[...]


def _make_table(emb_table, W, b):
    return pl.pallas_call(
        _mm_body,
        out_shape=jax.ShapeDtypeStruct((NUM_CHARS, NUM_CHARS), jnp.float32),
    )(emb_table, W, b.reshape(1, NUM_CHARS))


_NC, _NS = 2, 16
_NW = _NC * _NS

_B = 4096 * 20
_CH = 64
_BPW = _B // _NW
_NCHUNK = _BPW // _CH
_NPAIR = _NCHUNK // 2


def _gather_body(m_hbm, idx_hbm, out_hbm, idx_v, rows0, rows1,
                 g0, g1, s0, s1):
    wid = lax.axis_index("s") * _NC + lax.axis_index("c")
    base = wid * _BPW
    pltpu.sync_copy(idx_hbm.at[pl.ds(base, _BPW)], idx_v)

    def g_desc(buf, sem, c):
        return pltpu.make_async_copy(
            m_hbm.at[idx_v.at[pl.ds(c * _CH, _CH)]], buf, sem)

    def s_desc(buf, sem, c):
        return pltpu.make_async_copy(
            buf, out_hbm.at[pl.ds(base + c * _CH, _CH)], sem)

    g_desc(rows0, g0, 0).start()
    g_desc(rows1, g1, 1).start()

    def pair(p, carry):
        c0, c1 = 2 * p, 2 * p + 1
        g_desc(rows0, g0, c0).wait()
        s_desc(rows0, s0, c0).start()
        g_desc(rows1, g1, c1).wait()
        s_desc(rows1, s1, c1).start()
        s_desc(rows0, s0, c0).wait()
        @pl.when(p < _NPAIR - 1)
        def _():
            g_desc(rows0, g0, c0 + 2).start()
        s_desc(rows1, s1, c1).wait()
        @pl.when(p < _NPAIR - 1)
        def _():
            g_desc(rows1, g1, c1 + 2).start()
        return carry

    lax.fori_loop(0, _NPAIR, pair, 0)


@functools.lru_cache(maxsize=1)
def _gather_fn():
    return pl.kernel(
        _gather_body,
        mesh=plsc.VectorSubcoreMesh(core_axis_name="c", subcore_axis_name="s"),
        out_type=jax.ShapeDtypeStruct((_B, NUM_CHARS), jnp.float32),
        scratch_types=[
            pltpu.VMEM((_BPW,), jnp.int32),
            pltpu.VMEM((_CH, NUM_CHARS), jnp.float32),
            pltpu.VMEM((_CH, NUM_CHARS), jnp.float32),
            pltpu.SemaphoreType.DMA,
            pltpu.SemaphoreType.DMA,
            pltpu.SemaphoreType.DMA,
            pltpu.SemaphoreType.DMA,
        ],
        compiler_params=pltpu.CompilerParams(use_tc_tiling_on_sc=False),
    )


def kernel(x, emb_table, W, b):
    batch, seq = x.shape
    m = _make_table(emb_table, W, b)
    out = _gather_fn()(m, x.reshape(-1).astype(jnp.int32))
    return out.reshape(batch, seq, NUM_CHARS)
